# baseline (speedup 1.0000x reference)
import functools

import jax
import jax.numpy as jnp
from jax import lax
from jax.experimental import pallas as pl
from jax.experimental.pallas import tpu as pltpu
from jax.experimental.pallas import tpu_sc as plsc

VOCAB = 100000
EMB = 128
BATCH = 16384
NEG = 20

NUM_WORKERS = 32
CHUNK = 128

B_PER_W = BATCH // NUM_WORKERS
N_PER_W = BATCH * NEG // NUM_WORKERS

_mesh = plsc.VectorSubcoreMesh(core_axis_name="c", subcore_axis_name="s")


@functools.partial(
    pl.kernel,
    out_type=(
        jax.ShapeDtypeStruct((BATCH, EMB), jnp.float32),
        jax.ShapeDtypeStruct((BATCH, EMB), jnp.float32),
        jax.ShapeDtypeStruct((BATCH * NEG, EMB), jnp.float32),
    ),
    mesh=_mesh,
    scratch_types=[
        pltpu.VMEM((CHUNK,), jnp.int32),
        pltpu.VMEM((CHUNK, EMB), jnp.float32),
        pltpu.SemaphoreType.DMA,
    ],
)
def _sc_gather(center_hbm, context_hbm, cw_hbm, pw_hbm, nw_hbm,
               v_out, upos_out, uneg_out, idx_v, rows_v, sem):
    wid = lax.axis_index("s") * 2 + lax.axis_index("c")

    def gather_slice(table, idx_hbm, out_hbm, base, nchunks):
        @pl.loop(0, nchunks)
        def _(j):
            off = base + j * CHUNK
            pltpu.sync_copy(idx_hbm.at[pl.ds(off, CHUNK)], idx_v)
            pltpu.async_copy(table.at[idx_v], rows_v, sem).wait()
            pltpu.sync_copy(rows_v, out_hbm.at[pl.ds(off, CHUNK)])

    gather_slice(center_hbm, cw_hbm, v_out, wid * B_PER_W, B_PER_W // CHUNK)
    gather_slice(context_hbm, pw_hbm, upos_out, wid * B_PER_W, B_PER_W // CHUNK)
    gather_slice(context_hbm, nw_hbm, uneg_out, wid * N_PER_W, N_PER_W // CHUNK)


def _log_sigmoid(x):
    return jnp.minimum(x, 0.0) - jnp.log(1.0 + jnp.exp(-jnp.abs(x)))


BB = 512


def _dot_lastdim(x, ones):
    return lax.dot_general(x, ones, (((1,), (0,)), ((), ())),
                           precision=lax.Precision.HIGHEST)


def _loss_body(v_ref, upos_ref, uneg_ref, out_ref):
    i = pl.program_id(0)
    v = v_ref[...]
    ones = jnp.ones((EMB, 1), jnp.float32)
    acc = _log_sigmoid(_dot_lastdim(v * upos_ref[...], ones))
    for k in range(NEG):
        acc += _log_sigmoid(-_dot_lastdim(uneg_ref[:, k, :] * v, ones))
    block_sum = jnp.sum(acc).reshape(1, 1)

    @pl.when(i == 0)
    def _():
        out_ref[...] = jnp.zeros((1, 1), jnp.float32)

    out_ref[...] += block_sum

    @pl.when(i == pl.num_programs(0) - 1)
    def _():
        out_ref[...] = out_ref[...] * (-1.0 / BATCH)


_loss_call = pl.pallas_call(
    _loss_body,
    grid=(BATCH // BB,),
    in_specs=[
        pl.BlockSpec((BB, EMB), lambda i: (i, 0)),
        pl.BlockSpec((BB, EMB), lambda i: (i, 0)),
        pl.BlockSpec((BB, NEG, EMB), lambda i: (i, 0, 0)),
    ],
    out_specs=pl.BlockSpec((1, 1), lambda i: (0, 0)),
    out_shape=jax.ShapeDtypeStruct((1, 1), jnp.float32),
)


def kernel(center_emb, context_emb, center_words, pos_context_words, neg_context_words):
    cw = center_words.astype(jnp.int32)
    pw = pos_context_words.astype(jnp.int32)
    nw = neg_context_words.astype(jnp.int32).reshape(BATCH * NEG)
    v, u_pos, u_neg = _sc_gather(center_emb, context_emb, cw, pw, nw)
    loss = _loss_call(v, u_pos, u_neg.reshape(BATCH, NEG, EMB))
    return jnp.reshape(loss, ())


# speedup vs baseline: 2.1551x; 2.1551x over previous
import functools

import jax
import jax.numpy as jnp
from jax import lax
from jax.experimental import pallas as pl
from jax.experimental.pallas import tpu as pltpu
from jax.experimental.pallas import tpu_sc as plsc

VOCAB = 100000
EMB = 128
BATCH = 16384
NEG = 20

NUM_WORKERS = 32
CHUNK = 128

B_PER_W = BATCH // NUM_WORKERS
N_PER_W = BATCH * NEG // NUM_WORKERS

_mesh = plsc.VectorSubcoreMesh(core_axis_name="c", subcore_axis_name="s")


@functools.partial(
    pl.kernel,
    out_type=(
        jax.ShapeDtypeStruct((BATCH, EMB), jnp.float32),
        jax.ShapeDtypeStruct((BATCH, EMB), jnp.float32),
        jax.ShapeDtypeStruct((BATCH * NEG, EMB), jnp.float32),
    ),
    mesh=_mesh,
    scratch_types=[
        pltpu.VMEM((CHUNK,), jnp.int32),
        pltpu.VMEM((CHUNK, EMB), jnp.float32),
        pltpu.SemaphoreType.DMA,
    ],
)
def _sc_gather(center_hbm, context_hbm, cw_hbm, pw_hbm, nw_hbm,
               v_out, upos_out, uneg_out, idx_v, rows_v, sem):
    wid = lax.axis_index("s") * 2 + lax.axis_index("c")

    def gather_slice(table, idx_hbm, out_hbm, base, nchunks):
        @pl.loop(0, nchunks)
        def _(j):
            off = base + j * CHUNK
            pltpu.sync_copy(idx_hbm.at[pl.ds(off, CHUNK)], idx_v)
            pltpu.async_copy(table.at[idx_v], rows_v, sem).wait()
            pltpu.sync_copy(rows_v, out_hbm.at[pl.ds(off, CHUNK)])

    gather_slice(center_hbm, cw_hbm, v_out, wid * B_PER_W, B_PER_W // CHUNK)
    gather_slice(context_hbm, pw_hbm, upos_out, wid * B_PER_W, B_PER_W // CHUNK)
    gather_slice(context_hbm, nw_hbm, uneg_out, wid * N_PER_W, N_PER_W // CHUNK)


def _log_sigmoid(x):
    return jnp.minimum(x, 0.0) - jnp.log(1.0 + jnp.exp(-jnp.abs(x)))


BB = 512


def _loss_body(v_ref, upos_ref, uneg_ref, out_ref):
    i = pl.program_id(0)
    v = v_ref[...]
    pos = jnp.sum(v * upos_ref[...], axis=1)
    acc = _log_sigmoid(pos)
    for k in range(NEG):
        s = jnp.sum(uneg_ref[k] * v, axis=1)
        acc += _log_sigmoid(-s)
    block_sum = jnp.sum(acc).reshape(1, 1)

    @pl.when(i == 0)
    def _():
        out_ref[...] = jnp.zeros((1, 1), jnp.float32)

    out_ref[...] += block_sum

    @pl.when(i == pl.num_programs(0) - 1)
    def _():
        out_ref[...] = out_ref[...] * (-1.0 / BATCH)


_loss_call = pl.pallas_call(
    _loss_body,
    grid=(BATCH // BB,),
    in_specs=[
        pl.BlockSpec((BB, EMB), lambda i: (i, 0)),
        pl.BlockSpec((BB, EMB), lambda i: (i, 0)),
        pl.BlockSpec((NEG, BB, EMB), lambda i: (0, i, 0)),
    ],
    out_specs=pl.BlockSpec((1, 1), lambda i: (0, 0)),
    out_shape=jax.ShapeDtypeStruct((1, 1), jnp.float32),
)


def kernel(center_emb, context_emb, center_words, pos_context_words, neg_context_words):
    cw = center_words.astype(jnp.int32)
    pw = pos_context_words.astype(jnp.int32)
    nw = neg_context_words.astype(jnp.int32).T.reshape(BATCH * NEG)
    v, u_pos, u_neg = _sc_gather(center_emb, context_emb, cw, pw, nw)
    loss = _loss_call(v, u_pos, u_neg.reshape(NEG, BATCH, EMB))
    return jnp.reshape(loss, ())


# speedup vs baseline: 2.7509x; 1.2765x over previous
import functools

import jax
import jax.numpy as jnp
from jax import lax
from jax.experimental import pallas as pl
from jax.experimental.pallas import tpu as pltpu
from jax.experimental.pallas import tpu_sc as plsc

VOCAB = 100000
EMB = 128
BATCH = 16384
NEG = 20

NUM_WORKERS = 32
CHUNK = 128

B_PER_W = BATCH // NUM_WORKERS
N_PER_W = BATCH * NEG // NUM_WORKERS

_mesh = plsc.VectorSubcoreMesh(core_axis_name="c", subcore_axis_name="s")

NC_NEG = N_PER_W // CHUNK
NC_B = B_PER_W // CHUNK


@functools.partial(
    pl.kernel,
    out_type=(
        jax.ShapeDtypeStruct((BATCH, EMB), jnp.float32),
        jax.ShapeDtypeStruct((BATCH, EMB), jnp.float32),
        jax.ShapeDtypeStruct((BATCH * NEG, EMB), jnp.float32),
    ),
    mesh=_mesh,
    scratch_types=[
        pltpu.VMEM((NC_NEG, CHUNK), jnp.int32),
        pltpu.VMEM((CHUNK, EMB), jnp.float32),
        pltpu.VMEM((CHUNK, EMB), jnp.float32),
        pltpu.SemaphoreType.DMA,
        pltpu.SemaphoreType.DMA,
        pltpu.SemaphoreType.DMA,
        pltpu.SemaphoreType.DMA,
    ],
)
def _sc_gather(center_hbm, context_hbm, cw_hbm, pw_hbm, nw_hbm,
               v_out, upos_out, uneg_out,
               idx_v, rb0, rb1, sg0, sg1, sw0, sw1):
    wid = lax.axis_index("s") * 2 + lax.axis_index("c")

    def gather_slice(table, idx2d_hbm, out_hbm, chunk0, n):
        base = chunk0 * CHUNK
        pltpu.sync_copy(idx2d_hbm.at[pl.ds(chunk0, n)], idx_v.at[pl.ds(0, n)])

        def gst(j, rb, sem):
            pltpu.async_copy(table.at[idx_v.at[j]], rb, sem)

        def gwait(j, rb, sem):
            pltpu.make_async_copy(table.at[idx_v.at[j]], rb, sem).wait()

        def wst(j, rb, sem):
            pltpu.async_copy(rb, out_hbm.at[pl.ds(base + j * CHUNK, CHUNK)], sem)

        def wwait(j, rb, sem):
            pltpu.make_async_copy(
                rb, out_hbm.at[pl.ds(base + j * CHUNK, CHUNK)], sem).wait()

        gst(0, rb0, sg0)
        gwait(0, rb0, sg0)
        gst(1, rb1, sg1)
        wst(0, rb0, sw0)

        @pl.loop(0, (n - 2) // 2)
        def _(t):
            j1 = 1 + 2 * t
            gwait(j1, rb1, sg1)
            wwait(j1 - 1, rb0, sw0)
            gst(j1 + 1, rb0, sg0)
            wst(j1, rb1, sw1)
            j2 = 2 + 2 * t
            gwait(j2, rb0, sg0)
            wwait(j2 - 1, rb1, sw1)
            gst(j2 + 1, rb1, sg1)
            wst(j2, rb0, sw0)

        gwait(n - 1, rb1, sg1)
        wwait(n - 2, rb0, sw0)
        wst(n - 1, rb1, sw1)
        wwait(n - 1, rb1, sw1)

    gather_slice(center_hbm, cw_hbm, v_out, wid * NC_B, NC_B)
    gather_slice(context_hbm, pw_hbm, upos_out, wid * NC_B, NC_B)
    gather_slice(context_hbm, nw_hbm, uneg_out, wid * NC_NEG, NC_NEG)


def _log_sigmoid(x):
    return jnp.minimum(x, 0.0) - jnp.log(1.0 + jnp.exp(-jnp.abs(x)))


BB = 512


def _loss_body(v_ref, upos_ref, uneg_ref, out_ref):
    i = pl.program_id(0)
    v = v_ref[...]
    pos = jnp.sum(v * upos_ref[...], axis=1)
    acc = _log_sigmoid(pos)
    for k in range(NEG):
        s = jnp.sum(uneg_ref[k] * v, axis=1)
        acc += _log_sigmoid(-s)
    block_sum = jnp.sum(acc).reshape(1, 1)

    @pl.when(i == 0)
    def _():
        out_ref[...] = jnp.zeros((1, 1), jnp.float32)

    out_ref[...] += block_sum

    @pl.when(i == pl.num_programs(0) - 1)
    def _():
        out_ref[...] = out_ref[...] * (-1.0 / BATCH)


_loss_call = pl.pallas_call(
    _loss_body,
    grid=(BATCH // BB,),
    in_specs=[
        pl.BlockSpec((BB, EMB), lambda i: (i, 0)),
        pl.BlockSpec((BB, EMB), lambda i: (i, 0)),
        pl.BlockSpec((NEG, BB, EMB), lambda i: (0, i, 0)),
    ],
    out_specs=pl.BlockSpec((1, 1), lambda i: (0, 0)),
    out_shape=jax.ShapeDtypeStruct((1, 1), jnp.float32),
)


def kernel(center_emb, context_emb, center_words, pos_context_words, neg_context_words):
    cw = center_words.astype(jnp.int32).reshape(BATCH // CHUNK, CHUNK)
    pw = pos_context_words.astype(jnp.int32).reshape(BATCH // CHUNK, CHUNK)
    nw = neg_context_words.astype(jnp.int32).T.reshape(BATCH * NEG // CHUNK, CHUNK)
    v, u_pos, u_neg = _sc_gather(center_emb, context_emb, cw, pw, nw)
    loss = _loss_call(v, u_pos, u_neg.reshape(NEG, BATCH, EMB))
    return jnp.reshape(loss, ())


# speedup vs baseline: 2.8942x; 1.0521x over previous
import functools

import jax
import jax.numpy as jnp
from jax import lax
from jax.experimental import pallas as pl
from jax.experimental.pallas import tpu as pltpu
from jax.experimental.pallas import tpu_sc as plsc

VOCAB = 100000
EMB = 128
BATCH = 16384
NEG = 20

NUM_WORKERS = 32
CHUNK = 128

S = 4
BS = BATCH // S
NC_NEG = BS * NEG // (NUM_WORKERS * CHUNK)

_mesh = plsc.VectorSubcoreMesh(core_axis_name="c", subcore_axis_name="s")


@functools.partial(
    pl.kernel,
    out_type=(
        jax.ShapeDtypeStruct((BS, EMB), jnp.float32),
        jax.ShapeDtypeStruct((BS, EMB), jnp.float32),
        jax.ShapeDtypeStruct((BS * NEG, EMB), jnp.float32),
    ),
    mesh=_mesh,
    scratch_types=[
        pltpu.VMEM((NC_NEG + 4, CHUNK), jnp.int32),
        pltpu.VMEM((CHUNK, EMB), jnp.float32),
        pltpu.VMEM((CHUNK, EMB), jnp.float32),
        pltpu.SemaphoreType.DMA,
        pltpu.SemaphoreType.DMA,
        pltpu.SemaphoreType.DMA,
        pltpu.SemaphoreType.DMA,
    ],
)
def _sc_gather(center_hbm, context_hbm, cw_hbm, pw_hbm, nw_hbm,
               v_out, upos_out, uneg_out,
               idx_v, rb0, rb1, sg0, sg1, sw0, sw1):
    wid = lax.axis_index("s") * 2 + lax.axis_index("c")

    def gather_slice(table, idx2d_hbm, out_hbm, chunk0, n, nload):
        base = chunk0 * CHUNK
        a0 = pl.multiple_of((chunk0 // 8) * 8, 8)
        d = chunk0 - a0
        pltpu.sync_copy(idx2d_hbm.at[pl.ds(a0, nload)], idx_v.at[pl.ds(0, nload)])

        def gst(j, rb, sem):
            pltpu.async_copy(table.at[idx_v.at[d + j]], rb, sem)

        def gwait(j, rb, sem):
            pltpu.make_async_copy(table.at[idx_v.at[d + j]], rb, sem).wait()

        def wst(j, rb, sem):
            pltpu.async_copy(rb, out_hbm.at[pl.ds(base + j * CHUNK, CHUNK)], sem)

        def wwait(j, rb, sem):
            pltpu.make_async_copy(
                rb, out_hbm.at[pl.ds(base + j * CHUNK, CHUNK)], sem).wait()

        if n == 1:
            gst(0, rb0, sg0)
            gwait(0, rb0, sg0)
            wst(0, rb0, sw0)
            wwait(0, rb0, sw0)
            return

        gst(0, rb0, sg0)
        gwait(0, rb0, sg0)
        gst(1, rb1, sg1)
        wst(0, rb0, sw0)

        @pl.loop(0, (n - 2) // 2)
        def _(t):
            j1 = 1 + 2 * t
            gwait(j1, rb1, sg1)
            wwait(j1 - 1, rb0, sw0)
            gst(j1 + 1, rb0, sg0)
            wst(j1, rb1, sw1)
            j2 = 2 + 2 * t
            gwait(j2, rb0, sg0)
            wwait(j2 - 1, rb1, sw1)
            gst(j2 + 1, rb1, sg1)
            wst(j2, rb0, sw0)

        gwait(n - 1, rb1, sg1)
        wwait(n - 2, rb0, sw0)
        wst(n - 1, rb1, sw1)
        wwait(n - 1, rb1, sw1)

    gather_slice(center_hbm, cw_hbm, v_out, wid, 1, 8)
    gather_slice(context_hbm, pw_hbm, upos_out, wid, 1, 8)
    gather_slice(context_hbm, nw_hbm, uneg_out, wid * NC_NEG, NC_NEG, NC_NEG + 4)


def _log_sigmoid(x):
    return jnp.minimum(x, 0.0) - jnp.log(1.0 + jnp.exp(-jnp.abs(x)))


BB = 512


def _loss_body(v_ref, upos_ref, uneg_ref, out_ref):
    i = pl.program_id(0)
    v = v_ref[...]
    pos = jnp.sum(v * upos_ref[...], axis=1)
    acc = _log_sigmoid(pos)
    for k in range(NEG):
        s = jnp.sum(uneg_ref[k] * v, axis=1)
        acc += _log_sigmoid(-s)
    block_sum = jnp.sum(acc).reshape(1, 1)

    @pl.when(i == 0)
    def _():
        out_ref[...] = jnp.zeros((1, 1), jnp.float32)

    out_ref[...] += block_sum


_loss_call = pl.pallas_call(
    _loss_body,
    grid=(BS // BB,),
    in_specs=[
        pl.BlockSpec((BB, EMB), lambda i: (i, 0)),
        pl.BlockSpec((BB, EMB), lambda i: (i, 0)),
        pl.BlockSpec((NEG, BB, EMB), lambda i: (0, i, 0)),
    ],
    out_specs=pl.BlockSpec((1, 1), lambda i: (0, 0)),
    out_shape=jax.ShapeDtypeStruct((1, 1), jnp.float32),
)


def kernel(center_emb, context_emb, center_words, pos_context_words, neg_context_words):
    cw = center_words.astype(jnp.int32).reshape(S, NUM_WORKERS, CHUNK)
    pw = pos_context_words.astype(jnp.int32).reshape(S, NUM_WORKERS, CHUNK)
    nw = (neg_context_words.astype(jnp.int32).T
          .reshape(NEG, S, BS).transpose(1, 0, 2)
          .reshape(S, BS * NEG // CHUNK, CHUNK))
    total = jnp.zeros((1, 1), jnp.float32)
    for s in range(S):
        v, u_pos, u_neg = _sc_gather(center_emb, context_emb, cw[s], pw[s], nw[s])
        total = total + _loss_call(v, u_pos, u_neg.reshape(NEG, BS, EMB))
    return jnp.reshape(total * (-1.0 / BATCH), ())


# speedup vs baseline: 3.2064x; 1.1079x over previous
import functools

import jax
import jax.numpy as jnp
from jax import lax
from jax.experimental import pallas as pl
from jax.experimental.pallas import tpu as pltpu
from jax.experimental.pallas import tpu_sc as plsc

VOCAB = 100000
EMB = 128
BATCH = 16384
NEG = 20

NUM_WORKERS = 32
CHUNK = 128

S = 4
BS = BATCH // S
NC_NEG = BS * NEG // (NUM_WORKERS * CHUNK)

_mesh = plsc.VectorSubcoreMesh(core_axis_name="c", subcore_axis_name="s")


@functools.partial(
    pl.kernel,
    out_type=(
        jax.ShapeDtypeStruct((BS, EMB), jnp.float32),
        jax.ShapeDtypeStruct((BS, EMB), jnp.float32),
        jax.ShapeDtypeStruct((BS * NEG, EMB), jnp.float32),
    ),
    mesh=_mesh,
    scratch_types=[
        pltpu.VMEM((NC_NEG + 4, CHUNK), jnp.int32),
        pltpu.VMEM((CHUNK, EMB), jnp.float32),
        pltpu.VMEM((CHUNK, EMB), jnp.float32),
        pltpu.VMEM((CHUNK, EMB), jnp.float32),
        pltpu.VMEM((CHUNK, EMB), jnp.float32),
        pltpu.SemaphoreType.DMA,
        pltpu.SemaphoreType.DMA,
        pltpu.SemaphoreType.DMA,
        pltpu.SemaphoreType.DMA,
        pltpu.SemaphoreType.DMA,
        pltpu.SemaphoreType.DMA,
        pltpu.SemaphoreType.DMA,
        pltpu.SemaphoreType.DMA,
    ],
)
def _sc_gather(center_hbm, context_hbm, cw_hbm, pw_hbm, nw_hbm,
               v_out, upos_out, uneg_out,
               idx_v, rb0, rb1, rb2, rb3,
               sg0, sg1, sg2, sg3, sw0, sw1, sw2, sw3):
    wid = lax.axis_index("s") * 2 + lax.axis_index("c")
    rb = (rb0, rb1, rb2, rb3)
    sg = (sg0, sg1, sg2, sg3)
    sw = (sw0, sw1, sw2, sw3)

    def gather_slice(table, idx2d_hbm, out_hbm, chunk0, n, nload):
        base = chunk0 * CHUNK
        a0 = pl.multiple_of((chunk0 // 8) * 8, 8)
        d = chunk0 - a0
        pltpu.sync_copy(idx2d_hbm.at[pl.ds(a0, nload)], idx_v.at[pl.ds(0, nload)])

        def gst(j, b):
            pltpu.async_copy(table.at[idx_v.at[d + j]], rb[b], sg[b])

        def gwait(j, b):
            pltpu.make_async_copy(table.at[idx_v.at[d + j]], rb[b], sg[b]).wait()

        def wst(j, b):
            pltpu.async_copy(
                rb[b], out_hbm.at[pl.ds(base + j * CHUNK, CHUNK)], sw[b])

        def wwait(j, b):
            pltpu.make_async_copy(
                rb[b], out_hbm.at[pl.ds(base + j * CHUNK, CHUNK)], sw[b]).wait()

        if n == 1:
            gst(0, 0)
            gwait(0, 0)
            wst(0, 0)
            wwait(0, 0)
            return

        gst(0, 0)
        gst(1, 1)
        gst(2, 2)
        gwait(0, 0)
        gst(3, 3)
        wst(0, 0)

        @pl.loop(0, (n - 4) // 4)
        def _(t):
            for r in range(1, 5):
                j = r + 4 * t
                b = r % 4
                gwait(j, b)
                wwait(j - 1, (r - 1) % 4)
                gst(j + 3, (r + 3) % 4)
                wst(j, b)

        for r in range(3, 0, -1):
            j = n - r
            b = j % 4
            gwait(j, b)
            wwait(j - 1, (j - 1) % 4)
            wst(j, b)
        wwait(n - 1, (n - 1) % 4)

    gather_slice(center_hbm, cw_hbm, v_out, wid, 1, 8)
    gather_slice(context_hbm, pw_hbm, upos_out, wid, 1, 8)
    gather_slice(context_hbm, nw_hbm, uneg_out, wid * NC_NEG, NC_NEG, NC_NEG + 4)


def _log_sigmoid(x):
    return jnp.minimum(x, 0.0) - jnp.log(1.0 + jnp.exp(-jnp.abs(x)))


BB = 512


def _loss_body(v_ref, upos_ref, uneg_ref, out_ref):
    i = pl.program_id(0)
    v = v_ref[...]
    pos = jnp.sum(v * upos_ref[...], axis=1)
    acc = _log_sigmoid(pos)
    for k in range(NEG):
        s = jnp.sum(uneg_ref[k] * v, axis=1)
        acc += _log_sigmoid(-s)
    block_sum = jnp.sum(acc).reshape(1, 1)

    @pl.when(i == 0)
    def _():
        out_ref[...] = jnp.zeros((1, 1), jnp.float32)

    out_ref[...] += block_sum


_loss_call = pl.pallas_call(
    _loss_body,
    grid=(BS // BB,),
    in_specs=[
        pl.BlockSpec((BB, EMB), lambda i: (i, 0)),
        pl.BlockSpec((BB, EMB), lambda i: (i, 0)),
        pl.BlockSpec((NEG, BB, EMB), lambda i: (0, i, 0)),
    ],
    out_specs=pl.BlockSpec((1, 1), lambda i: (0, 0)),
    out_shape=jax.ShapeDtypeStruct((1, 1), jnp.float32),
)


def kernel(center_emb, context_emb, center_words, pos_context_words, neg_context_words):
    cw = center_words.astype(jnp.int32).reshape(S, NUM_WORKERS, CHUNK)
    pw = pos_context_words.astype(jnp.int32).reshape(S, NUM_WORKERS, CHUNK)
    nw = (neg_context_words.astype(jnp.int32).T
          .reshape(NEG, S, BS).transpose(1, 0, 2)
          .reshape(S, BS * NEG // CHUNK, CHUNK))
    total = jnp.zeros((1, 1), jnp.float32)
    for s in range(S):
        v, u_pos, u_neg = _sc_gather(center_emb, context_emb, cw[s], pw[s], nw[s])
        total = total + _loss_call(v, u_pos, u_neg.reshape(NEG, BS, EMB))
    return jnp.reshape(total * (-1.0 / BATCH), ())


# speedup vs baseline: 3.3294x; 1.0384x over previous
import functools

import jax
import jax.numpy as jnp
from jax import lax
from jax.experimental import pallas as pl
from jax.experimental.pallas import tpu as pltpu
from jax.experimental.pallas import tpu_sc as plsc

VOCAB = 100000
EMB = 128
BATCH = 16384
NEG = 20

NUM_WORKERS = 32
CHUNK = 128

S = 4
BS = BATCH // S
NC_NEG = BS * NEG // (NUM_WORKERS * CHUNK)

_mesh = plsc.VectorSubcoreMesh(core_axis_name="c", subcore_axis_name="s")


@functools.partial(
    pl.kernel,
    out_type=(
        jax.ShapeDtypeStruct((BS, EMB), jnp.float32),
        jax.ShapeDtypeStruct((BS, EMB), jnp.float32),
        jax.ShapeDtypeStruct((BS * NEG, EMB), jnp.float32),
    ),
    mesh=_mesh,
    scratch_types=[
        pltpu.VMEM((NC_NEG + 4, CHUNK), jnp.int32),
        pltpu.VMEM((CHUNK, EMB), jnp.float32),
        pltpu.VMEM((CHUNK, EMB), jnp.float32),
        pltpu.VMEM((CHUNK, EMB), jnp.float32),
        pltpu.VMEM((CHUNK, EMB), jnp.float32),
        pltpu.VMEM((CHUNK, EMB), jnp.float32),
        pltpu.VMEM((CHUNK, EMB), jnp.float32),
        pltpu.SemaphoreType.DMA,
        pltpu.SemaphoreType.DMA,
        pltpu.SemaphoreType.DMA,
        pltpu.SemaphoreType.DMA,
        pltpu.SemaphoreType.DMA,
        pltpu.SemaphoreType.DMA,
        pltpu.SemaphoreType.DMA,
        pltpu.SemaphoreType.DMA,
        pltpu.SemaphoreType.DMA,
        pltpu.SemaphoreType.DMA,
        pltpu.SemaphoreType.DMA,
        pltpu.SemaphoreType.DMA,
    ],
)
def _sc_gather(center_hbm, context_hbm, idx_hbm,
               v_out, upos_out, uneg_out,
               idx_v, rbc, rbp, rb0, rb1, rb2, rb3,
               sgc, sgp, swc, swp, sg0, sg1, sg2, sg3, sw0, sw1, sw2, sw3):
    wid = lax.axis_index("s") * 2 + lax.axis_index("c")
    rb = (rb0, rb1, rb2, rb3)
    sg = (sg0, sg1, sg2, sg3)
    sw = (sw0, sw1, sw2, sw3)
    n = NC_NEG

    pltpu.sync_copy(idx_hbm.at[pl.ds(wid * (NC_NEG + 4), NC_NEG + 4)], idx_v)

    pltpu.async_copy(center_hbm.at[idx_v.at[0]], rbc, sgc)
    pltpu.async_copy(context_hbm.at[idx_v.at[1]], rbp, sgp)

    base = wid * CHUNK

    def gst(j, b):
        pltpu.async_copy(context_hbm.at[idx_v.at[2 + j]], rb[b], sg[b])

    def gwait(j, b):
        pltpu.make_async_copy(context_hbm.at[idx_v.at[2 + j]], rb[b], sg[b]).wait()

    def wst(j, b):
        pltpu.async_copy(
            rb[b], uneg_out.at[pl.ds(base + j * BS, CHUNK)], sw[b])

    def wwait(j, b):
        pltpu.make_async_copy(
            rb[b], uneg_out.at[pl.ds(base + j * BS, CHUNK)], sw[b]).wait()

    gst(0, 0)
    gst(1, 1)
    gst(2, 2)
    gwait(0, 0)
    gst(3, 3)
    wst(0, 0)

    @pl.loop(0, (n - 4) // 4)
    def _(t):
        for r in range(1, 5):
            j = r + 4 * t
            b = r % 4
            gwait(j, b)
            wwait(j - 1, (r - 1) % 4)
            gst(j + 3, (r + 3) % 4)
            wst(j, b)

    for r in range(3, 0, -1):
        j = n - r
        b = j % 4
        gwait(j, b)
        wwait(j - 1, (j - 1) % 4)
        wst(j, b)

    pltpu.make_async_copy(center_hbm.at[idx_v.at[0]], rbc, sgc).wait()
    pltpu.async_copy(rbc, v_out.at[pl.ds(wid * CHUNK, CHUNK)], swc)
    pltpu.make_async_copy(context_hbm.at[idx_v.at[1]], rbp, sgp).wait()
    pltpu.async_copy(rbp, upos_out.at[pl.ds(wid * CHUNK, CHUNK)], swp)
    wwait(n - 1, (n - 1) % 4)
    pltpu.make_async_copy(rbc, v_out.at[pl.ds(wid * CHUNK, CHUNK)], swc).wait()
    pltpu.make_async_copy(rbp, upos_out.at[pl.ds(wid * CHUNK, CHUNK)], swp).wait()


def _log_sigmoid(x):
    return jnp.minimum(x, 0.0) - jnp.log(1.0 + jnp.exp(-jnp.abs(x)))


BB = 512


def _loss_body(v_ref, upos_ref, uneg_ref, out_ref):
    i = pl.program_id(0)
    v = v_ref[...]
    pos = jnp.sum(v * upos_ref[...], axis=1)
    acc = _log_sigmoid(pos)
    for k in range(NEG):
        s = jnp.sum(uneg_ref[k] * v, axis=1)
        acc += _log_sigmoid(-s)
    block_sum = jnp.sum(acc).reshape(1, 1)

    @pl.when(i == 0)
    def _():
        out_ref[...] = jnp.zeros((1, 1), jnp.float32)

    out_ref[...] += block_sum


_loss_call = pl.pallas_call(
    _loss_body,
    grid=(BS // BB,),
    in_specs=[
        pl.BlockSpec((BB, EMB), lambda i: (i, 0)),
        pl.BlockSpec((BB, EMB), lambda i: (i, 0)),
        pl.BlockSpec((NEG, BB, EMB), lambda i: (0, i, 0)),
    ],
    out_specs=pl.BlockSpec((1, 1), lambda i: (0, 0)),
    out_shape=jax.ShapeDtypeStruct((1, 1), jnp.float32),
)


def kernel(center_emb, context_emb, center_words, pos_context_words, neg_context_words):
    cw = center_words.astype(jnp.int32).reshape(S, NUM_WORKERS, 1, CHUNK)
    pw = pos_context_words.astype(jnp.int32).reshape(S, NUM_WORKERS, 1, CHUNK)
    nw = (neg_context_words.astype(jnp.int32).T
          .reshape(NEG, S, NUM_WORKERS, CHUNK).transpose(1, 2, 0, 3))
    pad = jnp.zeros((S, NUM_WORKERS, 2, CHUNK), jnp.int32)
    idx_all = jnp.concatenate([cw, pw, nw, pad], axis=2).reshape(
        S, NUM_WORKERS * (NC_NEG + 4), CHUNK)
    total = jnp.zeros((1, 1), jnp.float32)
    for s in range(S):
        v, u_pos, u_neg = _sc_gather(center_emb, context_emb, idx_all[s])
        total = total + _loss_call(v, u_pos, u_neg.reshape(NEG, BS, EMB))
    return jnp.reshape(total * (-1.0 / BATCH), ())


# speedup vs baseline: 3.4607x; 1.0394x over previous
import functools

import jax
import jax.numpy as jnp
from jax import lax
from jax.experimental import pallas as pl
from jax.experimental.pallas import tpu as pltpu
from jax.experimental.pallas import tpu_sc as plsc

VOCAB = 100000
EMB = 128
BATCH = 16384
NEG = 20

NUM_WORKERS = 32
CHUNK = 128

S = 4
BS = BATCH // S
NC_NEG = BS * NEG // (NUM_WORKERS * CHUNK)

_mesh = plsc.VectorSubcoreMesh(core_axis_name="c", subcore_axis_name="s")


@functools.partial(
    pl.kernel,
    out_type=(
        jax.ShapeDtypeStruct((BS, EMB), jnp.float32),
        jax.ShapeDtypeStruct((BS, EMB), jnp.float32),
        jax.ShapeDtypeStruct((BS * NEG, EMB), jnp.float32),
    ),
    mesh=_mesh,
    scratch_types=[
        pltpu.VMEM((NC_NEG + 4, CHUNK), jnp.int32),
        pltpu.VMEM((CHUNK, EMB), jnp.float32),
        pltpu.VMEM((CHUNK, EMB), jnp.float32),
        pltpu.VMEM((CHUNK, EMB), jnp.float32),
        pltpu.VMEM((CHUNK, EMB), jnp.float32),
        pltpu.VMEM((CHUNK, EMB), jnp.float32),
        pltpu.VMEM((CHUNK, EMB), jnp.float32),
        pltpu.SemaphoreType.DMA,
        pltpu.SemaphoreType.DMA,
        pltpu.SemaphoreType.DMA,
        pltpu.SemaphoreType.DMA,
        pltpu.SemaphoreType.DMA,
        pltpu.SemaphoreType.DMA,
        pltpu.SemaphoreType.DMA,
        pltpu.SemaphoreType.DMA,
        pltpu.SemaphoreType.DMA,
        pltpu.SemaphoreType.DMA,
        pltpu.SemaphoreType.DMA,
        pltpu.SemaphoreType.DMA,
    ],
)
def _sc_gather(center_hbm, context_hbm, idx_hbm,
               v_out, upos_out, uneg_out,
               idx_v, rbc, rbp, rb0, rb1, rb2, rb3,
               sgc, sgp, swc, swp, sg0, sg1, sg2, sg3, sw0, sw1, sw2, sw3):
    wid = lax.axis_index("s") * 2 + lax.axis_index("c")
    rb = (rb0, rb1, rb2, rb3)
    sg = (sg0, sg1, sg2, sg3)
    sw = (sw0, sw1, sw2, sw3)
    n = NC_NEG

    pltpu.sync_copy(idx_hbm.at[pl.ds(wid * (NC_NEG + 4), NC_NEG + 4)], idx_v)

    pltpu.async_copy(center_hbm.at[idx_v.at[0]], rbc, sgc)
    pltpu.async_copy(context_hbm.at[idx_v.at[1]], rbp, sgp)

    base = wid * CHUNK

    def gst(j, b):
        pltpu.async_copy(context_hbm.at[idx_v.at[2 + j]], rb[b], sg[b])

    def gwait(j, b):
        pltpu.make_async_copy(context_hbm.at[idx_v.at[2 + j]], rb[b], sg[b]).wait()

    def wst(j, b):
        pltpu.async_copy(
            rb[b], uneg_out.at[pl.ds(base + j * BS, CHUNK)], sw[b])

    def wwait(j, b):
        pltpu.make_async_copy(
            rb[b], uneg_out.at[pl.ds(base + j * BS, CHUNK)], sw[b]).wait()

    gst(0, 0)
    gst(1, 1)
    gst(2, 2)
    gwait(0, 0)
    gst(3, 3)
    wst(0, 0)

    @pl.loop(0, (n - 4) // 4)
    def _(t):
        for r in range(1, 5):
            j = r + 4 * t
            b = r % 4
            gwait(j, b)
            wwait(j - 1, (r - 1) % 4)
            gst(j + 3, (r + 3) % 4)
            wst(j, b)

    for r in range(3, 0, -1):
        j = n - r
        b = j % 4
        gwait(j, b)
        wwait(j - 1, (j - 1) % 4)
        wst(j, b)

    pltpu.make_async_copy(center_hbm.at[idx_v.at[0]], rbc, sgc).wait()
    pltpu.async_copy(rbc, v_out.at[pl.ds(wid * CHUNK, CHUNK)], swc)
    pltpu.make_async_copy(context_hbm.at[idx_v.at[1]], rbp, sgp).wait()
    pltpu.async_copy(rbp, upos_out.at[pl.ds(wid * CHUNK, CHUNK)], swp)
    wwait(n - 1, (n - 1) % 4)
    pltpu.make_async_copy(rbc, v_out.at[pl.ds(wid * CHUNK, CHUNK)], swc).wait()
    pltpu.make_async_copy(rbp, upos_out.at[pl.ds(wid * CHUNK, CHUNK)], swp).wait()


def _log_sigmoid(x):
    return jnp.minimum(x, 0.0) - jnp.log(1.0 + jnp.exp(-jnp.abs(x)))


BB = 512


def _loss_body(v_ref, upos_ref, uneg_ref, e_ref, out_ref):
    i = pl.program_id(0)
    v = v_ref[...]
    parts = [v * upos_ref[...]]
    for k in range(NEG):
        parts.append(uneg_ref[k] * v)
    z = jnp.concatenate(parts, axis=1).astype(jnp.bfloat16)
    scores = lax.dot_general(z, e_ref[...], (((1,), (0,)), ((), ())),
                             preferred_element_type=jnp.float32)
    block_sum = jnp.sum(_log_sigmoid(scores)).reshape(1, 1)

    @pl.when(i == 0)
    def _():
        out_ref[...] = jnp.zeros((1, 1), jnp.float32)

    out_ref[...] += block_sum


_loss_call = pl.pallas_call(
    _loss_body,
    grid=(BS // BB,),
    in_specs=[
        pl.BlockSpec((BB, EMB), lambda i: (i, 0)),
        pl.BlockSpec((BB, EMB), lambda i: (i, 0)),
        pl.BlockSpec((NEG, BB, EMB), lambda i: (0, i, 0)),
        pl.BlockSpec(((NEG + 1) * EMB, NEG + 1), lambda i: (0, 0)),
    ],
    out_specs=pl.BlockSpec((1, 1), lambda i: (0, 0)),
    out_shape=jax.ShapeDtypeStruct((1, 1), jnp.float32),
)


def _make_e():
    sign = jnp.concatenate([jnp.ones((1,)), -jnp.ones((NEG,))]).astype(jnp.float32)
    eye = jnp.repeat(jnp.eye(NEG + 1, dtype=jnp.float32), EMB, axis=0)
    return (eye * sign[None, :]).astype(jnp.bfloat16)


def kernel(center_emb, context_emb, center_words, pos_context_words, neg_context_words):
    cw = center_words.astype(jnp.int32).reshape(S, NUM_WORKERS, 1, CHUNK)
    pw = pos_context_words.astype(jnp.int32).reshape(S, NUM_WORKERS, 1, CHUNK)
    nw = (neg_context_words.astype(jnp.int32).T
          .reshape(NEG, S, NUM_WORKERS, CHUNK).transpose(1, 2, 0, 3))
    pad = jnp.zeros((S, NUM_WORKERS, 2, CHUNK), jnp.int32)
    idx_all = jnp.concatenate([cw, pw, nw, pad], axis=2).reshape(
        S, NUM_WORKERS * (NC_NEG + 4), CHUNK)
    e = _make_e()
    total = jnp.zeros((1, 1), jnp.float32)
    for s in range(S):
        v, u_pos, u_neg = _sc_gather(center_emb, context_emb, idx_all[s])
        total = total + _loss_call(v, u_pos, u_neg.reshape(NEG, BS, EMB), e)
    return jnp.reshape(total * (-1.0 / BATCH), ())


# speedup vs baseline: 3.4874x; 1.0077x over previous
import functools

import jax
import jax.numpy as jnp
from jax import lax
from jax.experimental import pallas as pl
from jax.experimental.pallas import tpu as pltpu
from jax.experimental.pallas import tpu_sc as plsc

VOCAB = 100000
EMB = 128
BATCH = 16384
NEG = 20

NUM_WORKERS = 32
CHUNK = 128

S = 4
BS = BATCH // S
NC_NEG = BS * NEG // (NUM_WORKERS * CHUNK)

_mesh = plsc.VectorSubcoreMesh(core_axis_name="c", subcore_axis_name="s")


@functools.partial(
    pl.kernel,
    out_type=(
        jax.ShapeDtypeStruct((BS, EMB), jnp.float32),
        jax.ShapeDtypeStruct((BS, EMB), jnp.float32),
        jax.ShapeDtypeStruct((BS * NEG, EMB), jnp.float32),
    ),
    mesh=_mesh,
    scratch_types=[
        pltpu.VMEM((NC_NEG + 4, CHUNK), jnp.int32),
        pltpu.VMEM((CHUNK, EMB), jnp.float32),
        pltpu.VMEM((CHUNK, EMB), jnp.float32),
        pltpu.VMEM((CHUNK, EMB), jnp.float32),
        pltpu.VMEM((CHUNK, EMB), jnp.float32),
        pltpu.VMEM((CHUNK, EMB), jnp.float32),
        pltpu.VMEM((CHUNK, EMB), jnp.float32),
        pltpu.SemaphoreType.DMA,
        pltpu.SemaphoreType.DMA,
        pltpu.SemaphoreType.DMA,
        pltpu.SemaphoreType.DMA,
        pltpu.SemaphoreType.DMA,
        pltpu.SemaphoreType.DMA,
        pltpu.SemaphoreType.DMA,
        pltpu.SemaphoreType.DMA,
        pltpu.SemaphoreType.DMA,
        pltpu.SemaphoreType.DMA,
        pltpu.SemaphoreType.DMA,
        pltpu.SemaphoreType.DMA,
    ],
)
def _sc_gather(center_hbm, context_hbm, idx_hbm,
               v_out, upos_out, uneg_out,
               idx_v, rbc, rbp, rb0, rb1, rb2, rb3,
               sgc, sgp, swc, swp, sg0, sg1, sg2, sg3, sw0, sw1, sw2, sw3):
    wid = lax.axis_index("s") * 2 + lax.axis_index("c")
    rb = (rb0, rb1, rb2, rb3)
    sg = (sg0, sg1, sg2, sg3)
    sw = (sw0, sw1, sw2, sw3)
    n = NC_NEG

    pltpu.sync_copy(idx_hbm.at[pl.ds(wid * (NC_NEG + 4), NC_NEG + 4)], idx_v)

    pltpu.async_copy(center_hbm.at[idx_v.at[0]], rbc, sgc)
    pltpu.async_copy(context_hbm.at[idx_v.at[1]], rbp, sgp)

    base = wid * CHUNK

    def gst(j, b):
        pltpu.async_copy(context_hbm.at[idx_v.at[2 + j]], rb[b], sg[b])

    def gwait(j, b):
        pltpu.make_async_copy(context_hbm.at[idx_v.at[2 + j]], rb[b], sg[b]).wait()

    def wst(j, b):
        pltpu.async_copy(
            rb[b], uneg_out.at[pl.ds(base + j * BS, CHUNK)], sw[b])

    def wwait(j, b):
        pltpu.make_async_copy(
            rb[b], uneg_out.at[pl.ds(base + j * BS, CHUNK)], sw[b]).wait()

    gst(0, 0)
    gst(1, 1)
    gst(2, 2)
    gwait(0, 0)
    gst(3, 3)
    wst(0, 0)

    @pl.loop(0, (n - 4) // 4)
    def _(t):
        for r in range(1, 5):
            j = r + 4 * t
            b = r % 4
            gwait(j, b)
            wwait(j - 1, (r - 1) % 4)
            gst(j + 3, (r + 3) % 4)
            wst(j, b)

    for r in range(3, 0, -1):
        j = n - r
        b = j % 4
        gwait(j, b)
        wwait(j - 1, (j - 1) % 4)
        wst(j, b)

    pltpu.make_async_copy(center_hbm.at[idx_v.at[0]], rbc, sgc).wait()
    pltpu.async_copy(rbc, v_out.at[pl.ds(wid * CHUNK, CHUNK)], swc)
    pltpu.make_async_copy(context_hbm.at[idx_v.at[1]], rbp, sgp).wait()
    pltpu.async_copy(rbp, upos_out.at[pl.ds(wid * CHUNK, CHUNK)], swp)
    wwait(n - 1, (n - 1) % 4)
    pltpu.make_async_copy(rbc, v_out.at[pl.ds(wid * CHUNK, CHUNK)], swc).wait()
    pltpu.make_async_copy(rbp, upos_out.at[pl.ds(wid * CHUNK, CHUNK)], swp).wait()


def _log_sigmoid(x):
    return jnp.minimum(x, 0.0) - jnp.log(1.0 + jnp.exp(-jnp.abs(x)))


BB = 1024


def _loss_body(v_ref, upos_ref, uneg_ref, e_ref, out_ref):
    i = pl.program_id(0)
    v = v_ref[...]
    parts = [v * upos_ref[...]]
    for k in range(NEG):
        parts.append(uneg_ref[k] * v)
    z = jnp.concatenate(parts, axis=1).astype(jnp.bfloat16)
    scores = lax.dot_general(z, e_ref[...], (((1,), (0,)), ((), ())),
                             preferred_element_type=jnp.float32)
    block_sum = jnp.sum(_log_sigmoid(scores)).reshape(1, 1)

    @pl.when(i == 0)
    def _():
        out_ref[...] = jnp.zeros((1, 1), jnp.float32)

    out_ref[...] += block_sum


_loss_call = pl.pallas_call(
    _loss_body,
    grid=(BS // BB,),
    in_specs=[
        pl.BlockSpec((BB, EMB), lambda i: (i, 0)),
        pl.BlockSpec((BB, EMB), lambda i: (i, 0)),
        pl.BlockSpec((NEG, BB, EMB), lambda i: (0, i, 0)),
        pl.BlockSpec(((NEG + 1) * EMB, NEG + 1), lambda i: (0, 0)),
    ],
    out_specs=pl.BlockSpec((1, 1), lambda i: (0, 0)),
    out_shape=jax.ShapeDtypeStruct((1, 1), jnp.float32),
)


def _make_e():
    sign = jnp.concatenate([jnp.ones((1,)), -jnp.ones((NEG,))]).astype(jnp.float32)
    eye = jnp.repeat(jnp.eye(NEG + 1, dtype=jnp.float32), EMB, axis=0)
    return (eye * sign[None, :]).astype(jnp.bfloat16)


def kernel(center_emb, context_emb, center_words, pos_context_words, neg_context_words):
    cw = center_words.astype(jnp.int32).reshape(S, NUM_WORKERS, 1, CHUNK)
    pw = pos_context_words.astype(jnp.int32).reshape(S, NUM_WORKERS, 1, CHUNK)
    nw = (neg_context_words.astype(jnp.int32).T
          .reshape(NEG, S, NUM_WORKERS, CHUNK).transpose(1, 2, 0, 3))
    pad = jnp.zeros((S, NUM_WORKERS, 2, CHUNK), jnp.int32)
    idx_all = jnp.concatenate([cw, pw, nw, pad], axis=2).reshape(
        S, NUM_WORKERS * (NC_NEG + 4), CHUNK)
    e = _make_e()
    total = jnp.zeros((1, 1), jnp.float32)
    for s in range(S):
        v, u_pos, u_neg = _sc_gather(center_emb, context_emb, idx_all[s])
        total = total + _loss_call(v, u_pos, u_neg.reshape(NEG, BS, EMB), e)
    return jnp.reshape(total * (-1.0 / BATCH), ())
